# baseline (speedup 1.0000x reference)
import functools

import jax
import jax.numpy as jnp
from jax import lax
from jax.experimental import pallas as pl
from jax.experimental.pallas import tpu as pltpu
from jax.experimental.pallas import tpu_sc as plsc

NE = 64
K = 2
BR = 2048
N = 16384
NWORK = 32
RPW = N // NWORK
L = 16


def _logits_block(x_ref, w_ref, b_ref, out_ref):
    out_ref[...] = lax.dot_general(
        w_ref[...], x_ref[...],
        dimension_numbers=(((1,), (1,)), ((), ())),
        preferred_element_type=jnp.float32,
    ) + b_ref[...]


def _tc_logits(x_flat, W, bcol):
    return pl.pallas_call(
        _logits_block,
        grid=(N // BR,),
        in_specs=[
            pl.BlockSpec((BR, x_flat.shape[1]), lambda i: (i, 0)),
            pl.BlockSpec((NE, x_flat.shape[1]), lambda i: (0, 0)),
            pl.BlockSpec((NE, 1), lambda i: (0, 0)),
        ],
        out_specs=pl.BlockSpec((NE, BR), lambda i: (0, i)),
        out_shape=jax.ShapeDtypeStruct((NE, N), jnp.float32),
        compiler_params=pltpu.CompilerParams(
            dimension_semantics=("parallel",),
        ),
    )(x_flat, W, bcol)


@functools.partial(
    pl.kernel,
    mesh=plsc.VectorSubcoreMesh(core_axis_name="c", subcore_axis_name="s"),
    out_type=[
        jax.ShapeDtypeStruct((K, N), jnp.float32),
        jax.ShapeDtypeStruct((K, N), jnp.int32),
    ],
    scratch_types=[
        pltpu.VMEM((NE, RPW), jnp.float32),
        pltpu.VMEM((K, RPW), jnp.float32),
        pltpu.VMEM((K, RPW), jnp.int32),
    ],
)
def _sc_top2(logits_hbm, wout_hbm, iout_hbm, slab, wv, iv):
    nc = 2
    wid = lax.axis_index("s") * nc + lax.axis_index("c")
    base = wid * RPW
    pltpu.sync_copy(logits_hbm.at[:, pl.ds(base, RPW)], slab)

    def jbody(j, carry):
        off = j * L
        m1 = jnp.full((L,), -jnp.inf, jnp.float32)
        m2 = jnp.full((L,), -jnp.inf, jnp.float32)
        i1 = jnp.zeros((L,), jnp.int32)
        i2 = jnp.zeros((L,), jnp.int32)
        for e in range(NE):
            v = slab[e, pl.ds(off, L)]
            ev = jnp.full((L,), e, jnp.int32)
            gt1 = v > m1
            gt2 = v > m2
            m2 = jnp.where(gt1, m1, jnp.where(gt2, v, m2))
            i2 = jnp.where(gt1, i1, jnp.where(gt2, ev, i2))
            m1 = jnp.where(gt1, v, m1)
            i1 = jnp.where(gt1, ev, i1)
        e2 = jnp.exp(m2 - m1)
        den = 1.0 + e2
        wv[0, pl.ds(off, L)] = 1.0 / den
        wv[1, pl.ds(off, L)] = e2 / den
        iv[0, pl.ds(off, L)] = i1
        iv[1, pl.ds(off, L)] = i2
        return carry

    lax.fori_loop(0, RPW // L, jbody, 0)
    pltpu.sync_copy(wv, wout_hbm.at[:, pl.ds(base, RPW)])
    pltpu.sync_copy(iv, iout_hbm.at[:, pl.ds(base, RPW)])


@jax.jit
def _router(x_flat, W, bcol):
    logits_t = _tc_logits(x_flat, W, bcol)
    w_t, i_t = _sc_top2(logits_t)
    return w_t, i_t


def kernel(x, W, b, training=False):
    batch, seq, hidden = x.shape
    x_flat = x.reshape(-1, hidden)
    w_t, i_t = _router(x_flat, W, b.reshape(NE, 1))
    top_k_weights = w_t.T.reshape(batch, seq, K)
    expert_indices = i_t.T.reshape(batch, seq, K)
    aux_loss = jnp.float32(0.0)
    return (top_k_weights, expert_indices, aux_loss)
